# Initial kernel scaffold; baseline (speedup 1.0000x reference)
import jax
import jax.numpy as jnp
from jax.experimental import pallas as pl


def kernel(x, Wr, br, We, be):
    raise NotImplementedError("write your pallas kernel here")


# speedup vs baseline: 4.6129x; 4.6129x over previous
import functools

import jax
import jax.numpy as jnp
from jax import lax
from jax.experimental import pallas as pl
from jax.experimental.pallas import tpu as pltpu

T = 4096
D = 768
E = 8
TB = 256


def _moe_block(x_ref, wr_ref, br_ref, we_ref, be_ref, out_ref):
    xb = x_ref[...]
    logits = jnp.dot(xb, wr_ref[...],
                     preferred_element_type=jnp.float32) + br_ref[...]
    m = jnp.max(logits, axis=1, keepdims=True)
    ex = jnp.exp(logits - m)
    probs = ex / jnp.sum(ex, axis=1, keepdims=True)

    iota = lax.broadcasted_iota(jnp.int32, (TB, E), 1)
    m1 = jnp.max(probs, axis=1, keepdims=True)
    idx1 = jnp.min(jnp.where(probs == m1, iota, E), axis=1, keepdims=True)
    sel1 = iota == idx1
    probs_m = jnp.where(sel1, -1.0, probs)
    m2 = jnp.max(probs_m, axis=1, keepdims=True)
    idx2 = jnp.min(jnp.where(probs_m == m2, iota, E), axis=1, keepdims=True)
    sel2 = iota == idx2
    w = jnp.where(sel1, m1, 0.0) + jnp.where(sel2, m2, 0.0)

    xb16 = xb.astype(jnp.bfloat16)
    acc = jnp.zeros((TB, D), dtype=jnp.float32)
    for e in range(E):
        ye = jnp.dot(xb16, we_ref[e], preferred_element_type=jnp.float32)
        acc += w[:, e:e + 1] * (ye + be_ref[e][None, :])
    out_ref[...] = acc


@jax.jit
def kernel(x, Wr, br, We, be):
    we16 = We.astype(jnp.bfloat16)
    grid = (T // TB,)
    return pl.pallas_call(
        _moe_block,
        grid=grid,
        in_specs=[
            pl.BlockSpec((TB, D), lambda b: (b, 0)),
            pl.BlockSpec((D, E), lambda b: (0, 0)),
            pl.BlockSpec((E,), lambda b: (0,)),
            pl.BlockSpec((E, D, D), lambda b: (0, 0, 0)),
            pl.BlockSpec((E, D), lambda b: (0, 0)),
        ],
        out_specs=pl.BlockSpec((TB, D), lambda b: (b, 0)),
        out_shape=jax.ShapeDtypeStruct((T, D), jnp.float32),
    )(x, Wr, br, we16, be)
